# Initial kernel scaffold; baseline (speedup 1.0000x reference)
import jax
import jax.numpy as jnp
from jax.experimental import pallas as pl


def kernel(x, table):
    raise NotImplementedError("write your pallas kernel here")


# speedup vs baseline: 1.1785x; 1.1785x over previous
import functools

import jax
import jax.numpy as jnp
from jax import lax
from jax.experimental import pallas as pl
from jax.experimental.pallas import tpu as pltpu
from jax.experimental.pallas import tpu_sc as plsc

NC = 2
NS = 16
NW = NC * NS

IDX_PER_STREAM = 128
STREAMS_PER_CHUNK = 20


def _sc_gather(nrows, emb):
  nstreams = nrows // IDX_PER_STREAM
  chunks_total = nstreams // STREAMS_PER_CHUNK
  assert chunks_total % NW == 0
  chunks_per_w = chunks_total // NW

  mesh = plsc.VectorSubcoreMesh(core_axis_name="c", subcore_axis_name="s")

  @functools.partial(
      pl.kernel,
      mesh=mesh,
      out_type=jax.ShapeDtypeStruct(
          (chunks_total, STREAMS_PER_CHUNK, IDX_PER_STREAM, emb),
          jnp.float32),
      scratch_types=[
          pltpu.VMEM((STREAMS_PER_CHUNK, IDX_PER_STREAM), jnp.int32),
          pltpu.VMEM((STREAMS_PER_CHUNK, IDX_PER_STREAM, emb), jnp.float32),
          pltpu.SemaphoreType.DMA,
      ],
      compiler_params=pltpu.CompilerParams(use_tc_tiling_on_sc=False),
  )
  def k(idx_hbm, table_hbm, out_hbm, idx_v, rows_v, sem):
    wid = lax.axis_index("s") * NC + lax.axis_index("c")

    def body(i, carry):
      c0 = wid * chunks_per_w + i
      pltpu.sync_copy(idx_hbm.at[c0], idx_v)
      copies = []
      for j in range(STREAMS_PER_CHUNK):
        copies.append(
            pltpu.async_copy(table_hbm.at[idx_v.at[j]], rows_v.at[j], sem))
      for c in copies:
        c.wait()
      pltpu.sync_copy(rows_v, out_hbm.at[c0])
      return carry

    lax.fori_loop(0, chunks_per_w, body, 0)

  return k


def _tc_transpose(b, l, e, g):
  def body(x_ref, o_ref):
    o_ref[...] = jnp.swapaxes(x_ref[...], 1, 2)

  return pl.pallas_call(
      body,
      grid=(b // g,),
      in_specs=[pl.BlockSpec((g, l, e), lambda i: (i, 0, 0))],
      out_specs=pl.BlockSpec((g, e, l), lambda i: (i, 0, 0)),
      out_shape=jax.ShapeDtypeStruct((b, e, l), jnp.float32),
  )


def kernel(x, table):
  b, _, l = x.shape
  v, e = table.shape
  nrows = b * l
  nchunks = nrows // (IDX_PER_STREAM * STREAMS_PER_CHUNK)
  idx3d = x.reshape(nchunks, STREAMS_PER_CHUNK, IDX_PER_STREAM)
  rows = _sc_gather(nrows, e)(idx3d, table)
  rows = rows.reshape(b, l, e)
  return _tc_transpose(b, l, e, 16)(rows)


# speedup vs baseline: 1.5626x; 1.3260x over previous
import functools

import jax
import jax.numpy as jnp
from jax import lax
from jax.experimental import pallas as pl
from jax.experimental.pallas import tpu as pltpu
from jax.experimental.pallas import tpu_sc as plsc

NC = 2
NS = 16
NW = NC * NS

SAMPLES_PER_CHUNK = 4
STREAMS_PER_CHUNK = 8
IDX_PER_STREAM = 100
LP = 200


def _sc_fused(b, l, e):
  chunks_total = b // SAMPLES_PER_CHUNK
  assert chunks_total % NW == 0
  chunks_per_w = chunks_total // NW

  mesh = plsc.VectorSubcoreMesh(core_axis_name="c", subcore_axis_name="s")

  @functools.partial(
      pl.kernel,
      mesh=mesh,
      out_type=jax.ShapeDtypeStruct((b * e * l,), jnp.float32),
      scratch_types=[
          pltpu.VMEM((STREAMS_PER_CHUNK, IDX_PER_STREAM), jnp.int32),
          pltpu.VMEM((SAMPLES_PER_CHUNK, l, e), jnp.float32),
          pltpu.VMEM((SAMPLES_PER_CHUNK * e * LP,), jnp.float32),
          pltpu.SemaphoreType.DMA,
      ],
      compiler_params=pltpu.CompilerParams(use_tc_tiling_on_sc=False, needs_layout_passes=False),
  )
  def k(idx_hbm, table_hbm, out_hbm, idx_v, rows_v, outt_v, sem):
    wid = lax.axis_index("s") * NC + lax.axis_index("c")
    iota = lax.iota(jnp.int32, 16)

    def body(i, carry):
      c0 = wid * chunks_per_w + i
      pltpu.sync_copy(idx_hbm.at[c0], idx_v)
      copies = []
      for j in range(STREAMS_PER_CHUNK):
        s, half = j // 2, j % 2
        copies.append(
            pltpu.async_copy(
                table_hbm.at[idx_v.at[j]],
                rows_v.at[s, pl.ds(half * IDX_PER_STREAM, IDX_PER_STREAM)],
                sem))
      for c in copies:
        c.wait()

      def tr_body(ll, cc):
        for s in range(SAMPLES_PER_CHUNK):
          base = s * e * LP + ll
          v0 = rows_v[s, ll, pl.ds(0, 16)]
          v1 = rows_v[s, ll, pl.ds(16, 16)]
          plsc.store_scatter(outt_v, [base + iota * LP], v0)
          plsc.store_scatter(outt_v, [base + (iota + 16) * LP], v1)
        return cc

      plsc.parallel_loop(0, l, 1, unroll=4, carry=jnp.int32(0))(tr_body)

      nwords = SAMPLES_PER_CHUNK * e * l
      pltpu.sync_copy(outt_v, out_hbm.at[pl.ds(c0 * nwords, nwords)])
      return carry

    lax.fori_loop(0, chunks_per_w, body, 0)

  return k


def kernel(x, table):
  b, _, l = x.shape
  v, e = table.shape
  idx3d = x.reshape(b // SAMPLES_PER_CHUNK, STREAMS_PER_CHUNK, IDX_PER_STREAM)
  flat = _sc_fused(b, l, e)(idx3d, table)
  return flat.reshape(b, e, l)


# speedup vs baseline: 1.6921x; 1.0829x over previous
import functools

import jax
import jax.numpy as jnp
from jax import lax
from jax.experimental import pallas as pl
from jax.experimental.pallas import tpu as pltpu
from jax.experimental.pallas import tpu_sc as plsc

NC = 2
NS = 16
NW = NC * NS

SPC = 4
NSTR = 8
IPS = 100


def _sc_fused(b, l, e, v):
  chunks_total = b // SPC
  assert chunks_total % NW == 0
  n = chunks_total // NW
  assert n % 2 == 0
  nwords = SPC * e * l

  mesh = plsc.VectorSubcoreMesh(core_axis_name="c", subcore_axis_name="s")

  @functools.partial(
      pl.kernel,
      mesh=mesh,
      out_type=jax.ShapeDtypeStruct((b * e * l,), jnp.float32),
      scratch_types=[
          pltpu.VMEM((2, NSTR, IPS), jnp.int32),
          pltpu.VMEM((2, SPC, l, e), jnp.float32),
          pltpu.VMEM((2, nwords), jnp.float32),
          pltpu.SemaphoreType.DMA,
          pltpu.SemaphoreType.DMA,
          pltpu.SemaphoreType.DMA,
          pltpu.SemaphoreType.DMA,
          pltpu.SemaphoreType.DMA,
          pltpu.SemaphoreType.DMA,
      ],
      compiler_params=pltpu.CompilerParams(
          use_tc_tiling_on_sc=False, needs_layout_passes=False),
  )
  def k(idx_hbm, table_hbm, out_hbm, idx_v, rows_v, outt_v, sg0, sg1, so0,
        so1, si0, si1):
    wid = lax.axis_index("s") * NC + lax.axis_index("c")
    iota = lax.iota(jnp.int32, 16)
    tbl = table_hbm
    idx3 = idx_hbm
    sg = (sg0, sg1)
    so = (so0, so1)
    si = (si0, si1)

    def idx_desc(c0, p):
      return pltpu.make_async_copy(idx3.at[c0], idx_v.at[p], si[p])

    def fire_streams(p):
      for j in range(NSTR):
        s, half = j // 2, j % 2
        pltpu.async_copy(
            tbl.at[idx_v.at[p, j]],
            rows_v.at[p, s, pl.ds(half * IPS, IPS)], sg[p])

    def drain_gather(p):
      for j in range(NSTR):
        s, half = j // 2, j % 2
        pltpu.make_async_copy(
            tbl.at[idx_v.at[p, j]],
            rows_v.at[p, s, pl.ds(half * IPS, IPS)], sg[p]).wait()

    def out_desc(c0, p):
      return pltpu.make_async_copy(
          outt_v.at[p], out_hbm.at[pl.ds(c0 * nwords, nwords)], so[p])

    idx_desc(wid * n, 0).start()
    idx_desc(wid * n, 0).wait()
    fire_streams(0)

    def body(i2, carry):
      for p in (0, 1):
        i = 2 * i2 + p
        c0 = wid * n + i

        @pl.when(i < n - 1)
        def _():
          idx_desc(c0 + 1, 1 - p).start()

        drain_gather(p)

        @pl.when(i < n - 1)
        def _():
          idx_desc(c0 + 1, 1 - p).wait()
          fire_streams(1 - p)

        @pl.when(i >= 2)
        def _():
          out_desc(c0 - 2, p).wait()

        def tr_body(ll, cc):
          for s in range(SPC):
            base = s * e * l + ll
            v0 = rows_v[p, s, ll, pl.ds(0, 16)]
            v1 = rows_v[p, s, ll, pl.ds(16, 16)]
            plsc.store_scatter(outt_v.at[p], [base + iota * l], v0)
            plsc.store_scatter(outt_v.at[p], [base + (iota + 16) * l], v1)
          return cc

        plsc.parallel_loop(0, l, 1, unroll=4, carry=jnp.int32(0))(tr_body)
        out_desc(c0, p).start()
      return carry

    lax.fori_loop(0, n // 2, body, 0)
    out_desc(wid * n + n - 2, 0).wait()
    out_desc(wid * n + n - 1, 1).wait()

  return k


def kernel(x, table):
  b, _, l = x.shape
  v, e = table.shape
  idx3 = x.reshape(b // SPC, NSTR, IPS)
  flat = _sc_fused(b, l, e, v)(idx3, table)
  return flat.reshape(b, e, l)
